# baseline (speedup 1.0000x reference)
import functools

import jax
import jax.numpy as jnp
from jax import lax
from jax.experimental import pallas as pl
from jax.experimental.pallas import tpu as pltpu
from jax.experimental.pallas import tpu_sc as plsc

ZMAXPAD = 88
DIM = 64
N_ATOMS = 1048576
HALF = N_ATOMS // 2
H2 = HALF // 2

NC = 2
NS = 16
NW = NC * NS
B_PER_W = N_ATOMS // NW
CHUNK = 128
N_CHUNKS = B_PER_W // CHUNK
NBUF = 4

BR = 256
BR2 = BR // 2


def _sc_gather_call():
    mesh = plsc.VectorSubcoreMesh(core_axis_name="c", subcore_axis_name="s")

    @functools.partial(
        pl.kernel,
        mesh=mesh,
        compiler_params=pltpu.CompilerParams(use_tc_tiling_on_sc=False),
        out_type=jax.ShapeDtypeStruct((N_ATOMS, DIM), jnp.float32),
        scratch_types=[
            pltpu.VMEM((B_PER_W,), jnp.int32),
            [pltpu.VMEM((CHUNK, DIM), jnp.float32) for _ in range(NBUF)],
            [pltpu.SemaphoreType.DMA for _ in range(NBUF)],
            [pltpu.SemaphoreType.DMA for _ in range(NBUF)],
        ],
    )
    def sc_gather(species_hbm, table_hbm, out_hbm, idx_v, rows, gsems, wsems):
        wid = lax.axis_index("s") * NC + lax.axis_index("c")
        base_row = wid * B_PER_W
        pltpu.sync_copy(species_hbm.at[pl.ds(base_row, B_PER_W)], idx_v)
        my_table = table_hbm.at[wid]

        def body(jj, _):
            base_j = NBUF * jj
            for k in range(NBUF):
                j = base_j + k

                @pl.when(jj > 0)
                def _drain_write():
                    pltpu.make_async_copy(
                        rows[k],
                        out_hbm.at[pl.ds(base_row + j * CHUNK, CHUNK), :],
                        wsems[k]).wait()

                pltpu.async_copy(
                    my_table.at[idx_v.at[pl.ds(j * CHUNK, CHUNK)]],
                    rows[k], gsems[k])

            for k in range(NBUF):
                j = base_j + k
                pltpu.make_async_copy(
                    my_table.at[idx_v.at[pl.ds(j * CHUNK, CHUNK)]],
                    rows[k], gsems[k]).wait()
                pltpu.async_copy(
                    rows[k],
                    out_hbm.at[pl.ds(base_row + j * CHUNK, CHUNK), :],
                    wsems[k])
            return None

        lax.fori_loop(0, N_CHUNKS // NBUF, body, None)
        for k in range(NBUF):
            j = N_CHUNKS - NBUF + k
            pltpu.make_async_copy(
                rows[k],
                out_hbm.at[pl.ds(base_row + j * CHUNK, CHUNK), :],
                wsems[k]).wait()

    return sc_gather


def _tc_format(y, e_mats, r_mats):
    def body(x0_ref, x1_ref, e0_ref, e1_ref, r0_ref, r1_ref, o_ref):
        xs = (x0_ref[...], x1_ref[...])
        es = (e0_ref[...], e1_ref[...])
        rs = (r0_ref[...], r1_ref[...])
        dn_t = (((0,), (0,)), ((), ()))
        dn = (((1,), (0,)), ((), ()))
        acc = None
        for p in range(2):
            op = None
            for q in range(2):
                z = xs[p][:, q * DIM:(q + 1) * DIM]
                t = lax.dot_general(z, es[q], dn_t,
                                    preferred_element_type=jnp.float32)
                op = t if op is None else op + t
            t2 = lax.dot_general(rs[p], op, dn,
                                 preferred_element_type=jnp.float32)
            acc = t2 if acc is None else acc + t2
        o_ref[...] = acc

    e0, e1 = e_mats
    r0, r1 = r_mats
    return pl.pallas_call(
        body,
        grid=(HALF // BR,),
        in_specs=[
            pl.BlockSpec((BR2, 128), lambda b: (b, 0)),
            pl.BlockSpec((BR2, 128), lambda b: (b + H2 // BR2, 0)),
            pl.BlockSpec((BR2, BR), lambda b: (0, 0)),
            pl.BlockSpec((BR2, BR), lambda b: (0, 0)),
            pl.BlockSpec((128, DIM), lambda b: (0, 0)),
            pl.BlockSpec((128, DIM), lambda b: (0, 0)),
        ],
        out_specs=pl.BlockSpec((128, BR), lambda b: (0, b)),
        out_shape=jax.ShapeDtypeStruct((128, HALF), jnp.float32),
    )(y, y, e0, e1, r0, r1)


def kernel(species, table):
    table_rep = jnp.tile(table[None], (NW, 1, 1))
    lin = _sc_gather_call()(species, table_rep)
    y = jnp.reshape(lin, (HALF, 128))

    cols = jnp.arange(BR)
    j = jnp.arange(BR2)
    e0 = (cols[None, :] == 2 * j[:, None]).astype(jnp.float32)
    e1 = (cols[None, :] == 2 * j[:, None] + 1).astype(jnp.float32)
    d = jnp.arange(DIM)
    k = jnp.arange(128)
    base_k = (d // 8) * 16 + (d % 8)
    r0 = (k[:, None] == base_k[None, :]).astype(jnp.float32)
    r1 = (k[:, None] == base_k[None, :] + 8).astype(jnp.float32)

    out_t2 = _tc_format(y, (e0, e1), (r0, r1))
    o = out_t2.reshape(8, 2, 8, HALF)
    o = o.transpose(1, 3, 0, 2)
    return o.reshape(N_ATOMS, DIM)


# speedup vs baseline: 1.4807x; 1.4807x over previous
import functools

import jax
import jax.numpy as jnp
from jax import lax
from jax.experimental import pallas as pl
from jax.experimental.pallas import tpu as pltpu
from jax.experimental.pallas import tpu_sc as plsc

ZMAXPAD = 88
DIM = 64
N_ATOMS = 1048576
HALF = N_ATOMS // 2
H2 = HALF // 2

NC = 2
NS = 16
NW = NC * NS
B_PER_W = N_ATOMS // NW
CHUNK = 128
N_CHUNKS = B_PER_W // CHUNK
NBUF = 4

BR = 512
BR2 = BR // 2


def _sc_gather_call():
    mesh = plsc.VectorSubcoreMesh(core_axis_name="c", subcore_axis_name="s")

    @functools.partial(
        pl.kernel,
        mesh=mesh,
        compiler_params=pltpu.CompilerParams(use_tc_tiling_on_sc=False),
        out_type=jax.ShapeDtypeStruct((N_ATOMS, DIM), jnp.float32),
        scratch_types=[
            pltpu.VMEM((B_PER_W,), jnp.int32),
            [pltpu.VMEM((CHUNK, DIM), jnp.float32) for _ in range(NBUF)],
            [pltpu.SemaphoreType.DMA for _ in range(NBUF)],
            [pltpu.SemaphoreType.DMA for _ in range(NBUF)],
        ],
    )
    def sc_gather(species_hbm, table_hbm, out_hbm, idx_v, rows, gsems, wsems):
        wid = lax.axis_index("s") * NC + lax.axis_index("c")
        base_row = wid * B_PER_W
        pltpu.sync_copy(species_hbm.at[pl.ds(base_row, B_PER_W)], idx_v)
        my_table = table_hbm.at[wid]

        def body(jj, _):
            base_j = NBUF * jj
            for k in range(NBUF):
                j = base_j + k

                @pl.when(jj > 0)
                def _drain_write():
                    pltpu.make_async_copy(
                        rows[k],
                        out_hbm.at[pl.ds(base_row + j * CHUNK, CHUNK), :],
                        wsems[k]).wait()

                pltpu.async_copy(
                    my_table.at[idx_v.at[pl.ds(j * CHUNK, CHUNK)]],
                    rows[k], gsems[k])

            for k in range(NBUF):
                j = base_j + k
                pltpu.make_async_copy(
                    my_table.at[idx_v.at[pl.ds(j * CHUNK, CHUNK)]],
                    rows[k], gsems[k]).wait()
                pltpu.async_copy(
                    rows[k],
                    out_hbm.at[pl.ds(base_row + j * CHUNK, CHUNK), :],
                    wsems[k])
            return None

        lax.fori_loop(0, N_CHUNKS // NBUF, body, None)
        for k in range(NBUF):
            j = N_CHUNKS - NBUF + k
            pltpu.make_async_copy(
                rows[k],
                out_hbm.at[pl.ds(base_row + j * CHUNK, CHUNK), :],
                wsems[k]).wait()

    return sc_gather


def _tc_format(y, e_mats, r_mats):
    def body(x0_ref, x1_ref, e0_ref, e1_ref, r0_ref, r1_ref, o_ref):
        xs = (x0_ref[...], x1_ref[...])
        es = (e0_ref[...], e1_ref[...])
        rs = (r0_ref[...], r1_ref[...])
        dn_t = (((0,), (0,)), ((), ()))
        dn = (((1,), (0,)), ((), ()))
        acc = None
        for p in range(2):
            op = None
            for q in range(2):
                z = xs[p][:, q * DIM:(q + 1) * DIM]
                t = lax.dot_general(z, es[q], dn_t,
                                    preferred_element_type=jnp.float32)
                op = t if op is None else op + t
            t2 = lax.dot_general(rs[p], op, dn,
                                 preferred_element_type=jnp.float32)
            acc = t2 if acc is None else acc + t2
        o_ref[...] = acc

    e0, e1 = e_mats
    r0, r1 = r_mats
    return pl.pallas_call(
        body,
        grid=(HALF // BR,),
        in_specs=[
            pl.BlockSpec((BR2, 128), lambda b: (b, 0)),
            pl.BlockSpec((BR2, 128), lambda b: (b + H2 // BR2, 0)),
            pl.BlockSpec((BR2, BR), lambda b: (0, 0)),
            pl.BlockSpec((BR2, BR), lambda b: (0, 0)),
            pl.BlockSpec((128, DIM), lambda b: (0, 0)),
            pl.BlockSpec((128, DIM), lambda b: (0, 0)),
        ],
        out_specs=pl.BlockSpec((128, BR), lambda b: (0, b)),
        out_shape=jax.ShapeDtypeStruct((128, HALF), jnp.float32),
        compiler_params=pltpu.CompilerParams(
            fuse_transposed_lhs_in_matmul=True),
    )(y, y, e0, e1, r0, r1)


def kernel(species, table):
    table_rep = jnp.tile(table[None], (NW, 1, 1))
    lin = _sc_gather_call()(species, table_rep)
    y = jnp.reshape(lin, (HALF, 128))

    cols = jnp.arange(BR)
    j = jnp.arange(BR2)
    e0 = (cols[None, :] == 2 * j[:, None]).astype(jnp.float32)
    e1 = (cols[None, :] == 2 * j[:, None] + 1).astype(jnp.float32)
    d = jnp.arange(DIM)
    k = jnp.arange(128)
    base_k = (d // 8) * 16 + (d % 8)
    r0 = (k[:, None] == base_k[None, :]).astype(jnp.float32)
    r1 = (k[:, None] == base_k[None, :] + 8).astype(jnp.float32)

    out_t2 = _tc_format(y, (e0, e1), (r0, r1))
    o = out_t2.reshape(8, 2, 8, HALF)
    o = o.transpose(1, 3, 0, 2)
    return o.reshape(N_ATOMS, DIM)


# speedup vs baseline: 1.8184x; 1.2281x over previous
import functools

import jax
import jax.numpy as jnp
from jax import lax
from jax.experimental import pallas as pl
from jax.experimental.pallas import tpu as pltpu
from jax.experimental.pallas import tpu_sc as plsc

ZMAXPAD = 88
DIM = 64
N_ATOMS = 1048576
HALF = N_ATOMS // 2

NC = 2
NS = 16
NW = NC * NS
B_PER_W = N_ATOMS // NW
CHUNK = 128
N_CHUNKS = B_PER_W // CHUNK
NBUF = 4

BR = 512


def _sc_gather_call():
    mesh = plsc.VectorSubcoreMesh(core_axis_name="c", subcore_axis_name="s")

    @functools.partial(
        pl.kernel,
        mesh=mesh,
        compiler_params=pltpu.CompilerParams(use_tc_tiling_on_sc=False),
        out_type=jax.ShapeDtypeStruct((HALF, 128), jnp.float32),
        scratch_types=[
            pltpu.VMEM((B_PER_W,), jnp.int32),
            [pltpu.VMEM((CHUNK, DIM), jnp.float32) for _ in range(NBUF)],
            [pltpu.SemaphoreType.DMA for _ in range(NBUF)],
            [pltpu.SemaphoreType.DMA for _ in range(NBUF)],
        ],
    )
    def sc_gather(species_hbm, table_hbm, out_hbm, idx_v, rows, gsems, wsems):
        wid = lax.axis_index("s") * NC + lax.axis_index("c")
        base_row = wid * B_PER_W
        pltpu.sync_copy(species_hbm.at[pl.ds(base_row, B_PER_W)], idx_v)
        my_table = table_hbm.at[wid]
        p = wid // (NW // 2)
        r_base = base_row - p * HALF
        c0 = p * DIM

        def body(jj, _):
            base_j = NBUF * jj
            for k in range(NBUF):
                j = base_j + k

                @pl.when(jj > 0)
                def _drain_write():
                    pltpu.make_async_copy(
                        rows[k],
                        out_hbm.at[pl.ds(r_base + j * CHUNK, CHUNK),
                                   pl.ds(c0, DIM)],
                        wsems[k]).wait()

                pltpu.async_copy(
                    my_table.at[idx_v.at[pl.ds(j * CHUNK, CHUNK)]],
                    rows[k], gsems[k])

            for k in range(NBUF):
                j = base_j + k
                pltpu.make_async_copy(
                    my_table.at[idx_v.at[pl.ds(j * CHUNK, CHUNK)]],
                    rows[k], gsems[k]).wait()
                pltpu.async_copy(
                    rows[k],
                    out_hbm.at[pl.ds(r_base + j * CHUNK, CHUNK),
                               pl.ds(c0, DIM)],
                    wsems[k])
            return None

        lax.fori_loop(0, N_CHUNKS // NBUF, body, None)
        for k in range(NBUF):
            j = N_CHUNKS - NBUF + k
            pltpu.make_async_copy(
                rows[k],
                out_hbm.at[pl.ds(r_base + j * CHUNK, CHUNK), pl.ds(c0, DIM)],
                wsems[k]).wait()

    return sc_gather


def _tc_format(y, perm):
    def body(x_ref, p_ref, o_ref):
        o_ref[...] = lax.dot_general(
            p_ref[...], x_ref[...], (((1,), (1,)), ((), ())),
            preferred_element_type=jnp.float32)

    return pl.pallas_call(
        body,
        grid=(HALF // BR,),
        in_specs=[
            pl.BlockSpec((BR, 128), lambda b: (b, 0)),
            pl.BlockSpec((128, 128), lambda b: (0, 0)),
        ],
        out_specs=pl.BlockSpec((128, BR), lambda b: (0, b)),
        out_shape=jax.ShapeDtypeStruct((128, HALF), jnp.float32),
    )(y, perm)


def kernel(species, table):
    table_rep = jnp.tile(table[None], (NW, 1, 1))
    y = _sc_gather_call()(species, table_rep)

    c = jnp.arange(128)
    k_of_c = ((c % DIM) // 8) * 16 + (c // DIM) * 8 + (c % 8)
    perm = (jnp.arange(128)[:, None] == k_of_c[None, :]).astype(jnp.float32)

    out_t2 = _tc_format(y, perm)
    o = out_t2.reshape(8, 2, 8, HALF)
    o = o.transpose(1, 3, 0, 2)
    return o.reshape(N_ATOMS, DIM)
